# Initial kernel scaffold; baseline (speedup 1.0000x reference)
import jax
import jax.numpy as jnp
from jax.experimental import pallas as pl


def kernel(x, edge_index, feat_imp, gat_w, gat_att_src, gat_att_dst, gat_bias, ego_w, ego_b, hop_w0, hop_b0, hop_w1, hop_b1, gate_w1, gate_b1, gate_w2, gate_b2, bn_gamma, bn_beta, bn_mean, bn_var, cls_w, cls_b):
    raise NotImplementedError("write your pallas kernel here")


# speedup vs baseline: 17.3848x; 17.3848x over previous
import functools

import jax
import jax.numpy as jnp
from jax import lax
from jax.experimental import pallas as pl
from jax.experimental.pallas import tpu as pltpu
from jax.experimental.pallas import tpu_sc as plsc

N_T = 10240
N1 = 10016
K = 128
NC = 2
NS = 16
RPT = N_T // NS
SROWS = 160
DROWS = 80


def _full16(v):
    return jnp.full((16,), v, jnp.int32)


def _tc1_body(n, x_ref, fi_ref, gw_ref, was_ref, wad_ref,
              xs_ref, xl_ref, as01_ref, as23_ref, ad01_ref, ad23_ref,
              pmax_ref):
    xs = x_ref[:] * jax.nn.sigmoid(fi_ref[:])
    xs_ref[:] = xs
    xl = jnp.dot(xs, gw_ref[:], preferred_element_type=jnp.float32)
    xl_ref[:] = xl
    asrc16 = jnp.dot(xl, was_ref[:], preferred_element_type=jnp.float32)
    ad16 = jnp.dot(xl, wad_ref[:], preferred_element_type=jnp.float32)
    as01_ref[:] = asrc16[:, 0:2]
    as23_ref[:] = asrc16[:, 2:4]
    ad01_ref[:] = ad16[:, 0:2]
    ad23_ref[:] = ad16[:, 2:4]
    pa = jnp.max(asrc16, axis=0, keepdims=True)
    pb = jnp.max(ad16, axis=0, keepdims=True)
    pmax_ref[0] = jnp.concatenate(
        [pa, pb, jnp.zeros((1, 96), jnp.float32)], axis=1)
    del n


def _sc1a_body(c1, asrc_s, adst_s, gmax_hbm, rowi_hbm, coli_hbm,
               w_out, s_out, deg_out,
               as_v, ad_v, s_acc, deg_acc, wall, rowi8, coli8, gv,
               idxb, idxb2, idxd, sp_s, sp_deg):
    cid = lax.axis_index("c")
    sid = lax.axis_index("s")
    i32 = jnp.int32
    z16 = jnp.zeros((16,), jnp.float32)
    iota16 = lax.iota(i32, 16)

    def _zs(r, c):
        for s in range(8):
            s_acc[r, pl.ds(s * 16, 16)] = z16
        return c
    lax.fori_loop(0, SROWS, _zs, 0)

    def _zd(r, c):
        for s in range(8):
            deg_acc[r, pl.ds(s * 16, 16)] = z16
        return c
    lax.fori_loop(0, DROWS, _zd, 0)

    @pl.when(sid < 10)
    def _():
        pltpu.sync_copy(s_acc.at[pl.ds(0, 16)],
                        sp_s.at[pl.ds(sid * 16, 16)])

    @pl.when(jnp.logical_and(cid == 0, sid < 10))
    def _():
        pltpu.sync_copy(deg_acc.at[pl.ds(0, 8)],
                        sp_deg.at[pl.ds(sid * 8, 8)])

    pltpu.sync_copy(asrc_s.at[cid], as_v)
    pltpu.sync_copy(adst_s.at[cid], ad_v)
    pltpu.sync_copy(gmax_hbm, gv)

    g0 = plsc.load_gather(gv, [_full16(2 * cid)])
    g1 = plsc.load_gather(gv, [_full16(2 * cid + 1)])
    mask2 = iota16 < 2
    maskl0 = iota16 < 1
    onesv = jnp.ones((16,), jnp.float32)

    def blk8(j8, carry):
        pltpu.sync_copy(rowi_hbm.at[sid, pl.ds(j8 * 8, 8)], rowi8)
        pltpu.sync_copy(coli_hbm.at[sid, pl.ds(j8 * 8, 8)], coli8)

        def chunk(jj, cc):
            j = j8 * 8 + jj

            def qloop(q, qc):
                rows16 = rowi8[jj, pl.ds(q * 16, 16)]
                cols16 = coli8[jj, pl.ds(q * 16, 16)]
                fr = rows16 * 2
                fc = cols16 * 2
                a0 = plsc.load_gather(as_v, [fr >> 7, fr & 127]) + \
                    plsc.load_gather(ad_v, [fc >> 7, fc & 127])
                a1 = plsc.load_gather(as_v, [(fr + 1) >> 7, (fr + 1) & 127]) \
                    + plsc.load_gather(ad_v, [(fc + 1) >> 7, (fc + 1) & 127])
                w0 = jnp.exp(jnp.maximum(a0, 0.2 * a0) - g0)
                w1 = jnp.exp(jnp.maximum(a1, 0.2 * a1) - g1)
                fw = (j * K + q * 16 + iota16) * 2
                plsc.store_scatter(wall, [fw >> 7, fw & 127], w0)
                fw1 = fw + 1
                plsc.store_scatter(wall, [fw1 >> 7, fw1 & 127], w1)
                return qc
            lax.fori_loop(0, K // 16, qloop, 0)

            def eloop(t, ec):
                for u in range(4):
                    e = t * 4 + u
                    ef = _full16(e)
                    jf = _full16(jj)
                    cv = plsc.load_gather(coli8, [jf, ef])
                    flat = cv * 2 + iota16
                    fw = _full16((j * K + e) * 2) + iota16
                    wv = plsc.load_gather(
                        wall, [fw >> 7, fw & 127], mask=mask2)
                    plsc.addupdate_scatter(
                        s_acc, [flat >> 7, flat & 127], wv, mask=mask2)
                return ec
            lax.fori_loop(0, K // 4, eloop, 0)

            @pl.when(cid == 0)
            def _():
                def dloop(t, dc):
                    for u in range(4):
                        e = t * 4 + u
                        rv = plsc.load_gather(
                            rowi8, [_full16(jj), _full16(e)])
                        plsc.addupdate_scatter(
                            deg_acc, [rv >> 7, rv & 127], onesv,
                            mask=maskl0)
                    return dc
                lax.fori_loop(0, K // 4, dloop, 0)
            return cc
        lax.fori_loop(0, 8, chunk, 0)
        return carry
    lax.fori_loop(0, c1 // 8, blk8, 0)

    pltpu.sync_copy(wall, w_out.at[cid, sid])
    plsc.subcore_barrier()

    def fill(buf, nrows, base):
        for kk in range(nrows // 16):
            buf[pl.ds(kk * 16, 16)] = iota16 + (base + kk * 16)
    fill(idxb, 128, 0)
    fill(idxb2, 32, 128)
    pltpu.sync_copy(s_acc.at[pl.ds(0, 128)], sp_s.at[idxb], add=True)
    pltpu.sync_copy(s_acc.at[pl.ds(128, 32)], sp_s.at[idxb2], add=True)

    @pl.when(cid == 0)
    def _():
        fill(idxd, DROWS, 0)
        pltpu.sync_copy(deg_acc, sp_deg.at[idxd], add=True)

    plsc.subcore_barrier()

    @pl.when(sid < 10)
    def _():
        pltpu.sync_copy(sp_s.at[pl.ds(sid * 16, 16)],
                        s_out.at[cid, pl.ds(sid * 16, 16)])

    @pl.when(jnp.logical_and(cid == 0, sid < 10))
    def _():
        pltpu.sync_copy(sp_deg.at[pl.ds(sid * 8, 8)],
                        deg_out.at[pl.ds(sid * 8, 8)])


def _sc1b_body(c1, xs_hbm, xl_hbm, w_hbm, rowi_hbm, coli_hbm,
               nsu_out,
               rowi8, coli8, dbuf, wch0, wch1, sem, acc):
    cid = lax.axis_index("c")
    sid = lax.axis_index("s")
    z16 = jnp.zeros((16,), jnp.float32)

    def _zrow(r, c):
        for s in range(8):
            dbuf[r, pl.ds(s * 16, 16)] = z16
        return c
    lax.fori_loop(0, K, _zrow, 0)
    for q in range(RPT // K):
        pltpu.sync_copy(dbuf, acc.at[pl.ds(sid * RPT + q * K, K)])
    plsc.subcore_barrier()

    @pl.when(cid == 0)
    def _():
        def blk8(j8, carry):
            pltpu.sync_copy(rowi_hbm.at[sid, pl.ds(j8 * 8, 8)], rowi8)
            pltpu.sync_copy(coli_hbm.at[sid, pl.ds(j8 * 8, 8)], coli8)

            def chunk(jj, cc):
                pltpu.async_copy(
                    xs_hbm.at[coli8.at[jj]], dbuf, sem).wait()
                pltpu.sync_copy(dbuf, acc.at[rowi8.at[jj]], add=True)
                return cc
            lax.fori_loop(0, 8, chunk, 0)
            return carry
        lax.fori_loop(0, c1 // 8, blk8, 0)

    @pl.when(cid != 0)
    def _():
        def blk8(j8, carry):
            pltpu.sync_copy(rowi_hbm.at[sid, pl.ds(j8 * 8, 8)], rowi8)
            pltpu.sync_copy(coli_hbm.at[sid, pl.ds(j8 * 8, 8)], coli8)

            def wgrp(g4, gc):
                pltpu.sync_copy(
                    w_hbm.at[0, sid, pl.ds((j8 * 2 + g4) * 8, 8)], wch0)
                pltpu.sync_copy(
                    w_hbm.at[1, sid, pl.ds((j8 * 2 + g4) * 8, 8)], wch1)

                def chunk(jj, cc):
                    lj = g4 * 4 + jj
                    d1 = pltpu.async_copy(
                        xl_hbm.at[rowi8.at[lj]], dbuf, sem)
                    d1.wait()

                    def edge(e, ec):
                        lf = _full16((jj * K + e) * 2)
                        for v in range(8):
                            fw = lf + ((v // 2) & 1)
                            sc = plsc.load_gather(
                                wch0 if v < 4 else wch1,
                                [fw >> 7, fw & 127])
                            dbuf[e, pl.ds(v * 16, 16)] = \
                                dbuf[e, pl.ds(v * 16, 16)] * sc
                        return ec
                    lax.fori_loop(0, K, edge, 0)
                    pltpu.sync_copy(dbuf, acc.at[coli8.at[lj]], add=True)
                    return cc
                lax.fori_loop(0, 4, chunk, 0)
                return gc
            lax.fori_loop(0, 2, wgrp, 0)
            return carry
        lax.fori_loop(0, c1 // 8, blk8, 0)

    plsc.subcore_barrier()
    sl = pl.ds(sid * RPT, RPT)
    pltpu.sync_copy(acc.at[sl], nsu_out.at[cid].at[sl])


def _sc1c_body(c2, h1_hbm, rowi_hbm, coli_hbm, ns2_out,
               rowi_v, coli_v, dbuf, sem, acc):
    cid = lax.axis_index("c")
    sid = lax.axis_index("s")
    wid = sid * NC + cid
    z16 = jnp.zeros((16,), jnp.float32)

    def _zrow(r, c):
        for s in range(8):
            dbuf[r, pl.ds(s * 16, 16)] = z16
        return c
    lax.fori_loop(0, K, _zrow, 0)
    for q in range(RPT // K):
        pltpu.sync_copy(dbuf, acc.at[pl.ds(sid * RPT + q * K, K)])
    plsc.subcore_barrier()

    pltpu.sync_copy(rowi_hbm.at[wid], rowi_v)
    pltpu.sync_copy(coli_hbm.at[wid], coli_v)

    def chunk(j, carry):
        pltpu.async_copy(h1_hbm.at[coli_v.at[j]], dbuf, sem).wait()
        pltpu.sync_copy(dbuf, acc.at[rowi_v.at[j]], add=True)
        return carry
    lax.fori_loop(0, c2, chunk, 0)

    plsc.subcore_barrier()
    sl = pl.ds(sid * RPT, RPT)
    pltpu.sync_copy(acc.at[sl], ns2_out.at[cid].at[sl])


def _tc2_body(ns_ref, deg_ref, s4_ref, as4_ref, ad4_ref, xs_ref, xl_ref,
              gmax_ref, gw1_ref, gb1_ref, gw2_ref, gb2_ref,
              ew_ref, hw_ref, bb_ref,
              h1_ref, ds_ref, g_ref, hh_ref, sb_ref, uself_ref):
    blk = ns_ref.shape[0]
    nsum = ns_ref[:]
    deg = deg_ref[:]
    deg_safe = jnp.maximum(deg, 1.0)
    h1 = nsum / deg_safe
    h1_ref[:] = h1
    ds_ref[:] = jnp.broadcast_to(1.0 / deg_safe, (blk, 128))

    xs = xs_ref[:]
    xn = xs / jnp.maximum(
        jnp.sqrt(jnp.sum(xs * xs, axis=1, keepdims=True)), 1e-12)
    mn = h1 / jnp.maximum(
        jnp.sqrt(jnp.sum(h1 * h1, axis=1, keepdims=True)), 1e-12)
    sim = jnp.sum(xn * mn, axis=1, keepdims=True)
    sim = jnp.where(deg > 0, sim, 1.0)
    delta = jax.nn.sigmoid(deg * (1.0 - sim) / 10.0 - 0.5)

    g16 = jax.nn.relu(delta * gw1_ref[:] + gb1_ref[:])
    gs = jnp.sum(g16 * gw2_ref[:], axis=1, keepdims=True) + gb2_ref[0, 0]
    g = jax.nn.sigmoid(gs)
    g_ref[:] = jnp.broadcast_to(g, (blk, 128))

    s = as4_ref[:] + ad4_ref[:]
    alpha = jnp.maximum(s, 0.2 * s)
    w_self = jnp.exp(alpha - gmax_ref[:, :4])
    s4 = s4_ref[:] + w_self
    rmat = (lax.broadcasted_iota(jnp.int32, (4, 128), 1) // 32
            == lax.broadcasted_iota(jnp.int32, (4, 128), 0)
            ).astype(jnp.float32)
    sb_ref[:] = jnp.dot(s4, rmat, preferred_element_type=jnp.float32)
    wb = jnp.dot(w_self, rmat, preferred_element_type=jnp.float32)
    uself_ref[:] = wb * xl_ref[:]

    hh = jnp.dot(xs, ew_ref[:], preferred_element_type=jnp.float32)
    hh = hh + jnp.dot(h1, hw_ref[:], preferred_element_type=jnp.float32)
    hh_ref[:] = jax.nn.relu(hh + bb_ref[:])


def _tc3_body(p0_ref, p1_ref, ds_ref, hh_ref, g_ref, u_ref, uself_ref,
              sb_ref, gbias_ref, hw1_ref, b1_ref,
              bng_ref, bnb_ref, bnm_ref, bnv_ref, clsw_ref, clsb_ref,
              out_ref):
    hlp = (u_ref[:] + uself_ref[:]) / (sb_ref[:] + 1e-16) + gbias_ref[:]
    h_low = jnp.where(hlp > 0, hlp, jnp.exp(jnp.minimum(hlp, 0.0)) - 1.0)
    h2 = (p0_ref[:] + p1_ref[:]) * ds_ref[:]
    f1 = jax.nn.relu(
        jnp.dot(h2, hw1_ref[:], preferred_element_type=jnp.float32)
        + b1_ref[:])
    h_high = hh_ref[:] + f1
    g = g_ref[:]
    h = (1.0 - g) * h_low + g * h_high
    h = (h - bnm_ref[:]) / jnp.sqrt(bnv_ref[:] + 1e-5) * bng_ref[:] \
        + bnb_ref[:]
    out_ref[:] = jnp.dot(h, clsw_ref[:], preferred_element_type=jnp.float32) \
        + clsb_ref[:]


def _row_spec(blk, w):
    return pl.BlockSpec((blk, w), lambda i: (i, 0))


def _full_spec(shape):
    return pl.BlockSpec(shape, lambda i: tuple(0 for _ in shape))


def kernel(x, edge_index, feat_imp, gat_w, gat_att_src, gat_att_dst,
           gat_bias, ego_w, ego_b, hop_w0, hop_b0, hop_w1, hop_b1,
           gate_w1, gate_b1, gate_w2, gate_b2,
           bn_gamma, bn_beta, bn_mean, bn_var, cls_w, cls_b):
    n = x.shape[0]
    e = edge_index.shape[1]
    f32 = jnp.float32

    x_pad = jnp.zeros((N_T, 128), f32).at[:n].set(x)
    lane128 = jnp.arange(128)
    was = jnp.zeros((128, 16), f32).at[lane128, lane128 // 32].set(
        gat_att_src.reshape(128))
    wad = jnp.zeros((128, 16), f32).at[lane128, lane128 // 32].set(
        gat_att_dst.reshape(128))

    row = edge_index[0]
    col = edge_index[1]
    c1 = -(-e // (NS * K))
    c1 = -(-c1 // 8) * 8
    e1 = c1 * NS * K
    rowi1 = jnp.concatenate([row, jnp.full((e1 - e,), n, jnp.int32)]
                            ).reshape(NS, c1, K)
    coli1 = jnp.concatenate([col, jnp.full((e1 - e,), n, jnp.int32)]
                            ).reshape(NS, c1, K)
    c2 = -(-e // (NC * NS * K))
    e2 = c2 * NC * NS * K
    rowi2 = jnp.concatenate([row, jnp.full((e2 - e,), n, jnp.int32)]
                            ).reshape(NC * NS, c2, K)
    coli2 = jnp.concatenate([col, jnp.full((e2 - e,), n, jnp.int32)]
                            ).reshape(NC * NS, c2, K)

    blk = 256
    grid = N_T // blk

    xs_t, xl_t, as01, as23, ad01, ad23, pmax = pl.pallas_call(
        functools.partial(_tc1_body, n),
        grid=(grid,),
        in_specs=[
            _row_spec(blk, 128),
            _full_spec((1, 128)),
            _full_spec((128, 128)),
            _full_spec((128, 16)),
            _full_spec((128, 16)),
        ],
        out_specs=[
            _row_spec(blk, 128),
            _row_spec(blk, 128),
            _row_spec(blk, 2),
            _row_spec(blk, 2),
            _row_spec(blk, 2),
            _row_spec(blk, 2),
            pl.BlockSpec((1, 1, 128), lambda i: (i, 0, 0)),
        ],
        out_shape=[
            jax.ShapeDtypeStruct((N_T, 128), f32),
            jax.ShapeDtypeStruct((N_T, 128), f32),
            jax.ShapeDtypeStruct((N_T, 2), f32),
            jax.ShapeDtypeStruct((N_T, 2), f32),
            jax.ShapeDtypeStruct((N_T, 2), f32),
            jax.ShapeDtypeStruct((N_T, 2), f32),
            jax.ShapeDtypeStruct((grid, 1, 128), f32),
        ],
    )(x_pad, feat_imp.reshape(1, 128), gat_w, was, wad)

    pm = jnp.max(pmax[:, 0, :], axis=0)
    gsum = pm[0:4] + pm[16:20]
    gmax4 = jnp.where(gsum > 0, gsum, 0.2 * gsum)
    gmax128 = jnp.zeros((128,), f32).at[:4].set(gmax4)
    asrc_s = jnp.stack([as01.reshape(SROWS, 128), as23.reshape(SROWS, 128)])
    adst_s = jnp.stack([ad01.reshape(SROWS, 128), ad23.reshape(SROWS, 128)])

    mesh = plsc.VectorSubcoreMesh(core_axis_name="c", subcore_axis_name="s")
    scp = pltpu.CompilerParams(needs_layout_passes=False)
    wrows = c1 * 2

    w_e, s_out, deg_out = pl.kernel(
        functools.partial(_sc1a_body, c1),
        out_type=[
            jax.ShapeDtypeStruct((NC, NS, wrows, 128), f32),
            jax.ShapeDtypeStruct((NC, SROWS, 128), f32),
            jax.ShapeDtypeStruct((DROWS, 128), f32),
        ],
        mesh=mesh,
        scratch_types=[
            pltpu.VMEM((SROWS, 128), f32),
            pltpu.VMEM((SROWS, 128), f32),
            pltpu.VMEM((SROWS, 128), f32),
            pltpu.VMEM((DROWS, 128), f32),
            pltpu.VMEM((wrows, 128), f32),
            pltpu.VMEM((8, K), jnp.int32),
            pltpu.VMEM((8, K), jnp.int32),
            pltpu.VMEM((128,), f32),
            pltpu.VMEM((128,), jnp.int32),
            pltpu.VMEM((32,), jnp.int32),
            pltpu.VMEM((DROWS,), jnp.int32),
            pltpu.VMEM_SHARED((SROWS, 128), f32),
            pltpu.VMEM_SHARED((DROWS, 128), f32),
        ],
        compiler_params=scp,
    )(asrc_s, adst_s, gmax128, rowi1, coli1)

    s4 = jnp.concatenate(
        [s_out[0].reshape(N_T, 2), s_out[1].reshape(N_T, 2)], axis=1)
    deg = deg_out.reshape(N_T, 1)

    nsu = pl.kernel(
        functools.partial(_sc1b_body, c1),
        out_type=jax.ShapeDtypeStruct((NC, N_T, 128), f32),
        mesh=mesh,
        scratch_types=[
            pltpu.VMEM((8, K), jnp.int32),
            pltpu.VMEM((8, K), jnp.int32),
            pltpu.VMEM((K, 128), f32),
            pltpu.VMEM((8, 128), f32),
            pltpu.VMEM((8, 128), f32),
            pltpu.SemaphoreType.DMA,
            pltpu.VMEM_SHARED((N_T, 128), f32),
        ],
        compiler_params=scp,
    )(xs_t, xl_t, w_e, rowi1, coli1)
    ns, u_acc = nsu[0], nsu[1]

    gw1p = jnp.zeros((1, 128), f32).at[0, :16].set(gate_w1[0])
    gb1p = jnp.zeros((1, 128), f32).at[0, :16].set(gate_b1)
    gw2p = jnp.zeros((1, 128), f32).at[0, :16].set(gate_w2[:, 0])
    ew = jnp.zeros((128, 128), f32).at[:, :42].set(ego_w)
    hw = jnp.zeros((128, 128), f32).at[:, 42:84].set(hop_w0)
    bb = jnp.zeros((1, 128), f32).at[0, :42].set(ego_b).at[0, 42:84].set(
        hop_b0)
    as4 = jnp.concatenate([as01, as23], axis=1)
    ad4 = jnp.concatenate([ad01, ad23], axis=1)

    h1t, dsb, gb, hh, sb, uself = pl.pallas_call(
        _tc2_body,
        grid=(grid,),
        in_specs=[
            _row_spec(blk, 128),
            _row_spec(blk, 1),
            _row_spec(blk, 4),
            _row_spec(blk, 4),
            _row_spec(blk, 4),
            _row_spec(blk, 128),
            _row_spec(blk, 128),
            _full_spec((1, 16)),
            _full_spec((1, 128)),
            _full_spec((1, 128)),
            _full_spec((1, 128)),
            _full_spec((1, 1)),
            _full_spec((128, 128)),
            _full_spec((128, 128)),
            _full_spec((1, 128)),
        ],
        out_specs=[_row_spec(blk, 128)] * 6,
        out_shape=[jax.ShapeDtypeStruct((N_T, 128), f32)] * 6,
    )(ns, deg, s4, as4, ad4, xs_t, xl_t, gmax128[:16].reshape(1, 16),
      gw1p, gb1p, gw2p, gate_b2.reshape(1, 1), ew, hw, bb)

    ns2 = pl.kernel(
        functools.partial(_sc1c_body, c2),
        out_type=jax.ShapeDtypeStruct((NC, N_T, 128), f32),
        mesh=mesh,
        scratch_types=[
            pltpu.VMEM((c2, K), jnp.int32),
            pltpu.VMEM((c2, K), jnp.int32),
            pltpu.VMEM((K, 128), f32),
            pltpu.SemaphoreType.DMA,
            pltpu.VMEM_SHARED((N_T, 128), f32),
        ],
        compiler_params=scp,
    )(h1t, rowi2, coli2)

    hw1 = jnp.zeros((128, 128), f32).at[:, 84:128].set(hop_w1)
    b1p = jnp.zeros((1, 128), f32).at[0, 84:128].set(hop_b1)
    clswp = jnp.zeros((128, 128), f32).at[:, :2].set(cls_w)
    clsbp = jnp.zeros((1, 128), f32).at[0, :2].set(cls_b)

    out = pl.pallas_call(
        _tc3_body,
        grid=(grid,),
        in_specs=[
            _row_spec(blk, 128),
            _row_spec(blk, 128),
            _row_spec(blk, 128),
            _row_spec(blk, 128),
            _row_spec(blk, 128),
            _row_spec(blk, 128),
            _row_spec(blk, 128),
            _row_spec(blk, 128),
            _full_spec((1, 128)),
            _full_spec((128, 128)),
            _full_spec((1, 128)),
            _full_spec((1, 128)),
            _full_spec((1, 128)),
            _full_spec((1, 128)),
            _full_spec((1, 128)),
            _full_spec((128, 128)),
            _full_spec((1, 128)),
        ],
        out_specs=_row_spec(blk, 128),
        out_shape=jax.ShapeDtypeStruct((N_T, 128), f32),
    )(ns2[0], ns2[1], dsb, hh, gb, u_acc, uself, sb,
      gat_bias.reshape(1, 128), hw1, b1p,
      bn_gamma.reshape(1, 128), bn_beta.reshape(1, 128),
      bn_mean.reshape(1, 128), bn_var.reshape(1, 128), clswp, clsbp)

    return out[:n, :2]


# speedup vs baseline: 23.9820x; 1.3795x over previous
import functools

import jax
import jax.numpy as jnp
from jax import lax
from jax.experimental import pallas as pl
from jax.experimental.pallas import tpu as pltpu
from jax.experimental.pallas import tpu_sc as plsc

N_T = 10240
N1 = 10016
K = 128
NC = 2
NS = 16
RPT = N_T // NS
SROWS = 160
DROWS = 80


def _full16(v):
    return jnp.full((16,), v, jnp.int32)


def _tc1_body(n, x_ref, fi_ref, gw_ref, was_ref, wad_ref,
              xs_ref, xl_ref, as01_ref, as23_ref, ad01_ref, ad23_ref,
              pmax_ref):
    xs = x_ref[:] * jax.nn.sigmoid(fi_ref[:])
    xs_ref[:] = xs
    xl = jnp.dot(xs, gw_ref[:], preferred_element_type=jnp.float32)
    xl_ref[:] = xl
    asrc16 = jnp.dot(xl, was_ref[:], preferred_element_type=jnp.float32)
    ad16 = jnp.dot(xl, wad_ref[:], preferred_element_type=jnp.float32)
    as01_ref[:] = asrc16[:, 0:2]
    as23_ref[:] = asrc16[:, 2:4]
    ad01_ref[:] = ad16[:, 0:2]
    ad23_ref[:] = ad16[:, 2:4]
    pa = jnp.max(asrc16, axis=0, keepdims=True)
    pb = jnp.max(ad16, axis=0, keepdims=True)
    pmax_ref[0] = jnp.concatenate(
        [pa, pb, jnp.zeros((1, 96), jnp.float32)], axis=1)
    del n


def _sc1a_body(c1, asrc_s, adst_s, gmax_hbm, rowi_hbm, coli_hbm,
               w_out, s_out, deg_out,
               as_v, ad_v, s_acc, deg_acc, wall, rowi8, coli8, gv,
               idxb, idxb2, idxd, sp_s, sp_deg):
    cid = lax.axis_index("c")
    sid = lax.axis_index("s")
    i32 = jnp.int32
    z16 = jnp.zeros((16,), jnp.float32)
    iota16 = lax.iota(i32, 16)

    def _zs(r, c):
        for s in range(8):
            s_acc[r, pl.ds(s * 16, 16)] = z16
        return c
    lax.fori_loop(0, SROWS, _zs, 0)

    def _zd(r, c):
        for s in range(8):
            deg_acc[r, pl.ds(s * 16, 16)] = z16
        return c
    lax.fori_loop(0, DROWS, _zd, 0)

    @pl.when(sid < 10)
    def _():
        pltpu.sync_copy(s_acc.at[pl.ds(0, 16)],
                        sp_s.at[pl.ds(sid * 16, 16)])

    @pl.when(jnp.logical_and(cid == 0, sid < 10))
    def _():
        pltpu.sync_copy(deg_acc.at[pl.ds(0, 8)],
                        sp_deg.at[pl.ds(sid * 8, 8)])

    pltpu.sync_copy(asrc_s.at[cid], as_v)
    pltpu.sync_copy(adst_s.at[cid], ad_v)
    pltpu.sync_copy(gmax_hbm, gv)

    g0 = plsc.load_gather(gv, [_full16(2 * cid)])
    g1 = plsc.load_gather(gv, [_full16(2 * cid + 1)])
    mask2 = iota16 < 2
    maskl0 = iota16 < 1
    onesv = jnp.ones((16,), jnp.float32)

    def blk8(j8, carry):
        pltpu.sync_copy(rowi_hbm.at[sid, pl.ds(j8 * 8, 8)], rowi8)
        pltpu.sync_copy(coli_hbm.at[sid, pl.ds(j8 * 8, 8)], coli8)

        def chunk(jj, cc):
            j = j8 * 8 + jj

            def qloop(q, qc):
                rows16 = rowi8[jj, pl.ds(q * 16, 16)]
                cols16 = coli8[jj, pl.ds(q * 16, 16)]
                fr = rows16 * 2
                fc = cols16 * 2
                a0 = plsc.load_gather(as_v, [fr >> 7, fr & 127]) + \
                    plsc.load_gather(ad_v, [fc >> 7, fc & 127])
                a1 = plsc.load_gather(as_v, [(fr + 1) >> 7, (fr + 1) & 127]) \
                    + plsc.load_gather(ad_v, [(fc + 1) >> 7, (fc + 1) & 127])
                w0 = jnp.exp(jnp.maximum(a0, 0.2 * a0) - g0)
                w1 = jnp.exp(jnp.maximum(a1, 0.2 * a1) - g1)
                fw = (j * K + q * 16 + iota16) * 2
                plsc.store_scatter(wall, [fw >> 7, fw & 127], w0)
                fw1 = fw + 1
                plsc.store_scatter(wall, [fw1 >> 7, fw1 & 127], w1)
                return qc
            lax.fori_loop(0, K // 16, qloop, 0)

            def eloop(t, ec):
                for u in range(4):
                    e = t * 4 + u
                    ef = _full16(e)
                    jf = _full16(jj)
                    cv = plsc.load_gather(coli8, [jf, ef])
                    flat = cv * 2 + iota16
                    fw = _full16((j * K + e) * 2) + iota16
                    wv = plsc.load_gather(
                        wall, [fw >> 7, fw & 127], mask=mask2)
                    plsc.addupdate_scatter(
                        s_acc, [flat >> 7, flat & 127], wv, mask=mask2)
                return ec
            lax.fori_loop(0, K // 4, eloop, 0)

            @pl.when(cid == 0)
            def _():
                def dloop(t, dc):
                    for u in range(4):
                        e = t * 4 + u
                        rv = plsc.load_gather(
                            rowi8, [_full16(jj), _full16(e)])
                        plsc.addupdate_scatter(
                            deg_acc, [rv >> 7, rv & 127], onesv,
                            mask=maskl0)
                    return dc
                lax.fori_loop(0, K // 4, dloop, 0)
            return cc
        lax.fori_loop(0, 8, chunk, 0)
        return carry
    lax.fori_loop(0, c1 // 8, blk8, 0)

    pltpu.sync_copy(wall, w_out.at[cid, sid])
    plsc.subcore_barrier()

    def fill(buf, nrows, base):
        for kk in range(nrows // 16):
            buf[pl.ds(kk * 16, 16)] = iota16 + (base + kk * 16)
    fill(idxb, 128, 0)
    fill(idxb2, 32, 128)
    pltpu.sync_copy(s_acc.at[pl.ds(0, 128)], sp_s.at[idxb], add=True)
    pltpu.sync_copy(s_acc.at[pl.ds(128, 32)], sp_s.at[idxb2], add=True)

    @pl.when(cid == 0)
    def _():
        fill(idxd, DROWS, 0)
        pltpu.sync_copy(deg_acc, sp_deg.at[idxd], add=True)

    plsc.subcore_barrier()

    @pl.when(sid < 10)
    def _():
        pltpu.sync_copy(sp_s.at[pl.ds(sid * 16, 16)],
                        s_out.at[cid, pl.ds(sid * 16, 16)])

    @pl.when(jnp.logical_and(cid == 0, sid < 10))
    def _():
        pltpu.sync_copy(sp_deg.at[pl.ds(sid * 8, 8)],
                        deg_out.at[pl.ds(sid * 8, 8)])


def _sc1b_body(c1, xs_hbm, xl_hbm, w_hbm, rowi_hbm, coli_hbm,
               nsu_out,
               rowi8, coli8, dbuf0, dbuf1, wch0, wch1, sem0, sem1, acc):
    cid = lax.axis_index("c")
    sid = lax.axis_index("s")
    z16 = jnp.zeros((16,), jnp.float32)
    bufs = (dbuf0, dbuf1)
    sems = (sem0, sem1)

    def _zrow(r, c):
        for s in range(8):
            dbuf0[r, pl.ds(s * 16, 16)] = z16
        return c
    lax.fori_loop(0, K, _zrow, 0)
    for q in range(RPT // K):
        pltpu.sync_copy(dbuf0, acc.at[pl.ds(sid * RPT + q * K, K)])
    plsc.subcore_barrier()

    @pl.when(cid == 0)
    def _():
        def blk8(j8, carry):
            pltpu.sync_copy(rowi_hbm.at[sid, pl.ds(j8 * 8, 8)], rowi8)
            pltpu.sync_copy(coli_hbm.at[sid, pl.ds(j8 * 8, 8)], coli8)
            d = pltpu.async_copy(xs_hbm.at[coli8.at[0]], bufs[0], sems[0])
            for jj in range(8):
                p = jj & 1
                if jj < 7:
                    dn = pltpu.async_copy(
                        xs_hbm.at[coli8.at[jj + 1]], bufs[1 - p],
                        sems[1 - p])
                d.wait()
                pltpu.sync_copy(bufs[p], acc.at[rowi8.at[jj]], add=True)
                if jj < 7:
                    d = dn
            return carry
        lax.fori_loop(0, c1 // 8, blk8, 0)

    @pl.when(cid != 0)
    def _():
        def blk8(j8, carry):
            pltpu.sync_copy(rowi_hbm.at[sid, pl.ds(j8 * 8, 8)], rowi8)
            pltpu.sync_copy(coli_hbm.at[sid, pl.ds(j8 * 8, 8)], coli8)
            d = pltpu.async_copy(xl_hbm.at[rowi8.at[0]], bufs[0], sems[0])
            for jj in range(8):
                p = jj & 1
                if jj % 4 == 0:
                    g4 = jj // 4
                    pltpu.sync_copy(
                        w_hbm.at[0, sid, pl.ds((j8 * 2 + g4) * 8, 8)],
                        wch0)
                    pltpu.sync_copy(
                        w_hbm.at[1, sid, pl.ds((j8 * 2 + g4) * 8, 8)],
                        wch1)
                if jj < 7:
                    dn = pltpu.async_copy(
                        xl_hbm.at[rowi8.at[jj + 1]], bufs[1 - p],
                        sems[1 - p])
                d.wait()
                db = bufs[p]
                ljk = (jj % 4) * K

                def edge(t, ec):
                    for u in range(2):
                        e = t * 2 + u
                        lf = _full16((ljk + e) * 2)
                        s0 = plsc.load_gather(wch0, [lf >> 7, lf & 127])
                        lf1 = lf + 1
                        s1 = plsc.load_gather(wch0, [lf1 >> 7, lf1 & 127])
                        s2 = plsc.load_gather(wch1, [lf >> 7, lf & 127])
                        s3 = plsc.load_gather(wch1, [lf1 >> 7, lf1 & 127])
                        for v, sc in ((0, s0), (1, s0), (2, s1), (3, s1),
                                      (4, s2), (5, s2), (6, s3), (7, s3)):
                            db[e, pl.ds(v * 16, 16)] = \
                                db[e, pl.ds(v * 16, 16)] * sc
                    return ec
                lax.fori_loop(0, K // 2, edge, 0)
                pltpu.sync_copy(db, acc.at[coli8.at[jj]], add=True)
                if jj < 7:
                    d = dn
            return carry
        lax.fori_loop(0, c1 // 8, blk8, 0)

    plsc.subcore_barrier()
    sl = pl.ds(sid * RPT, RPT)
    pltpu.sync_copy(acc.at[sl], nsu_out.at[cid].at[sl])


def _sc1c_body(c2, h1_hbm, rowi_hbm, coli_hbm, ns2_out,
               rowi8, coli8, dbuf0, dbuf1, sem0, sem1, acc):
    cid = lax.axis_index("c")
    sid = lax.axis_index("s")
    wid = sid * NC + cid
    z16 = jnp.zeros((16,), jnp.float32)
    bufs = (dbuf0, dbuf1)
    sems = (sem0, sem1)

    def _zrow(r, c):
        for s in range(8):
            dbuf0[r, pl.ds(s * 16, 16)] = z16
        return c
    lax.fori_loop(0, K, _zrow, 0)
    for q in range(RPT // K):
        pltpu.sync_copy(dbuf0, acc.at[pl.ds(sid * RPT + q * K, K)])
    plsc.subcore_barrier()

    def blk8(j8, carry):
        pltpu.sync_copy(rowi_hbm.at[wid, pl.ds(j8 * 8, 8)], rowi8)
        pltpu.sync_copy(coli_hbm.at[wid, pl.ds(j8 * 8, 8)], coli8)
        d = pltpu.async_copy(h1_hbm.at[coli8.at[0]], bufs[0], sems[0])
        for jj in range(8):
            p = jj & 1
            if jj < 7:
                dn = pltpu.async_copy(
                    h1_hbm.at[coli8.at[jj + 1]], bufs[1 - p], sems[1 - p])
            d.wait()
            pltpu.sync_copy(bufs[p], acc.at[rowi8.at[jj]], add=True)
            if jj < 7:
                d = dn
        return carry
    lax.fori_loop(0, c2 // 8, blk8, 0)

    plsc.subcore_barrier()
    sl = pl.ds(sid * RPT, RPT)
    pltpu.sync_copy(acc.at[sl], ns2_out.at[cid].at[sl])


def _tc2_body(ns_ref, deg_ref, s4_ref, as4_ref, ad4_ref, xs_ref, xl_ref,
              gmax_ref, gw1_ref, gb1_ref, gw2_ref, gb2_ref,
              ew_ref, hw_ref, bb_ref,
              h1_ref, ds_ref, g_ref, hh_ref, sb_ref, uself_ref):
    blk = ns_ref.shape[0]
    nsum = ns_ref[:]
    deg = deg_ref[:]
    deg_safe = jnp.maximum(deg, 1.0)
    h1 = nsum / deg_safe
    h1_ref[:] = h1
    ds_ref[:] = jnp.broadcast_to(1.0 / deg_safe, (blk, 128))

    xs = xs_ref[:]
    xn = xs / jnp.maximum(
        jnp.sqrt(jnp.sum(xs * xs, axis=1, keepdims=True)), 1e-12)
    mn = h1 / jnp.maximum(
        jnp.sqrt(jnp.sum(h1 * h1, axis=1, keepdims=True)), 1e-12)
    sim = jnp.sum(xn * mn, axis=1, keepdims=True)
    sim = jnp.where(deg > 0, sim, 1.0)
    delta = jax.nn.sigmoid(deg * (1.0 - sim) / 10.0 - 0.5)

    g16 = jax.nn.relu(delta * gw1_ref[:] + gb1_ref[:])
    gs = jnp.sum(g16 * gw2_ref[:], axis=1, keepdims=True) + gb2_ref[0, 0]
    g = jax.nn.sigmoid(gs)
    g_ref[:] = jnp.broadcast_to(g, (blk, 128))

    s = as4_ref[:] + ad4_ref[:]
    alpha = jnp.maximum(s, 0.2 * s)
    w_self = jnp.exp(alpha - gmax_ref[:, :4])
    s4 = s4_ref[:] + w_self
    rmat = (lax.broadcasted_iota(jnp.int32, (4, 128), 1) // 32
            == lax.broadcasted_iota(jnp.int32, (4, 128), 0)
            ).astype(jnp.float32)
    sb_ref[:] = jnp.dot(s4, rmat, preferred_element_type=jnp.float32)
    wb = jnp.dot(w_self, rmat, preferred_element_type=jnp.float32)
    uself_ref[:] = wb * xl_ref[:]

    hh = jnp.dot(xs, ew_ref[:], preferred_element_type=jnp.float32)
    hh = hh + jnp.dot(h1, hw_ref[:], preferred_element_type=jnp.float32)
    hh_ref[:] = jax.nn.relu(hh + bb_ref[:])


def _tc3_body(p0_ref, p1_ref, ds_ref, hh_ref, g_ref, u_ref, uself_ref,
              sb_ref, gbias_ref, hw1_ref, b1_ref,
              bng_ref, bnb_ref, bnm_ref, bnv_ref, clsw_ref, clsb_ref,
              out_ref):
    hlp = (u_ref[:] + uself_ref[:]) / (sb_ref[:] + 1e-16) + gbias_ref[:]
    h_low = jnp.where(hlp > 0, hlp, jnp.exp(jnp.minimum(hlp, 0.0)) - 1.0)
    h2 = (p0_ref[:] + p1_ref[:]) * ds_ref[:]
    f1 = jax.nn.relu(
        jnp.dot(h2, hw1_ref[:], preferred_element_type=jnp.float32)
        + b1_ref[:])
    h_high = hh_ref[:] + f1
    g = g_ref[:]
    h = (1.0 - g) * h_low + g * h_high
    h = (h - bnm_ref[:]) / jnp.sqrt(bnv_ref[:] + 1e-5) * bng_ref[:] \
        + bnb_ref[:]
    out_ref[:] = jnp.dot(h, clsw_ref[:], preferred_element_type=jnp.float32) \
        + clsb_ref[:]


def _row_spec(blk, w):
    return pl.BlockSpec((blk, w), lambda i: (i, 0))


def _full_spec(shape):
    return pl.BlockSpec(shape, lambda i: tuple(0 for _ in shape))


def kernel(x, edge_index, feat_imp, gat_w, gat_att_src, gat_att_dst,
           gat_bias, ego_w, ego_b, hop_w0, hop_b0, hop_w1, hop_b1,
           gate_w1, gate_b1, gate_w2, gate_b2,
           bn_gamma, bn_beta, bn_mean, bn_var, cls_w, cls_b):
    n = x.shape[0]
    e = edge_index.shape[1]
    f32 = jnp.float32

    x_pad = jnp.zeros((N_T, 128), f32).at[:n].set(x)
    lane128 = jnp.arange(128)
    was = jnp.zeros((128, 16), f32).at[lane128, lane128 // 32].set(
        gat_att_src.reshape(128))
    wad = jnp.zeros((128, 16), f32).at[lane128, lane128 // 32].set(
        gat_att_dst.reshape(128))

    row = edge_index[0]
    col = edge_index[1]
    c1 = -(-e // (NS * K))
    c1 = -(-c1 // 8) * 8
    e1 = c1 * NS * K
    rowi1 = jnp.concatenate([row, jnp.full((e1 - e,), n, jnp.int32)]
                            ).reshape(NS, c1, K)
    coli1 = jnp.concatenate([col, jnp.full((e1 - e,), n, jnp.int32)]
                            ).reshape(NS, c1, K)
    c2 = -(-e // (NC * NS * K))
    c2 = -(-c2 // 8) * 8
    e2 = c2 * NC * NS * K
    rowi2 = jnp.concatenate([row, jnp.full((e2 - e,), n, jnp.int32)]
                            ).reshape(NC * NS, c2, K)
    coli2 = jnp.concatenate([col, jnp.full((e2 - e,), n, jnp.int32)]
                            ).reshape(NC * NS, c2, K)

    blk = 256
    grid = N_T // blk

    xs_t, xl_t, as01, as23, ad01, ad23, pmax = pl.pallas_call(
        functools.partial(_tc1_body, n),
        grid=(grid,),
        in_specs=[
            _row_spec(blk, 128),
            _full_spec((1, 128)),
            _full_spec((128, 128)),
            _full_spec((128, 16)),
            _full_spec((128, 16)),
        ],
        out_specs=[
            _row_spec(blk, 128),
            _row_spec(blk, 128),
            _row_spec(blk, 2),
            _row_spec(blk, 2),
            _row_spec(blk, 2),
            _row_spec(blk, 2),
            pl.BlockSpec((1, 1, 128), lambda i: (i, 0, 0)),
        ],
        out_shape=[
            jax.ShapeDtypeStruct((N_T, 128), f32),
            jax.ShapeDtypeStruct((N_T, 128), f32),
            jax.ShapeDtypeStruct((N_T, 2), f32),
            jax.ShapeDtypeStruct((N_T, 2), f32),
            jax.ShapeDtypeStruct((N_T, 2), f32),
            jax.ShapeDtypeStruct((N_T, 2), f32),
            jax.ShapeDtypeStruct((grid, 1, 128), f32),
        ],
    )(x_pad, feat_imp.reshape(1, 128), gat_w, was, wad)

    pm = jnp.max(pmax[:, 0, :], axis=0)
    gsum = pm[0:4] + pm[16:20]
    gmax4 = jnp.where(gsum > 0, gsum, 0.2 * gsum)
    gmax128 = jnp.zeros((128,), f32).at[:4].set(gmax4)
    asrc_s = jnp.stack([as01.reshape(SROWS, 128), as23.reshape(SROWS, 128)])
    adst_s = jnp.stack([ad01.reshape(SROWS, 128), ad23.reshape(SROWS, 128)])

    mesh = plsc.VectorSubcoreMesh(core_axis_name="c", subcore_axis_name="s")
    scp = pltpu.CompilerParams(needs_layout_passes=False)
    wrows = c1 * 2

    w_e, s_out, deg_out = pl.kernel(
        functools.partial(_sc1a_body, c1),
        out_type=[
            jax.ShapeDtypeStruct((NC, NS, wrows, 128), f32),
            jax.ShapeDtypeStruct((NC, SROWS, 128), f32),
            jax.ShapeDtypeStruct((DROWS, 128), f32),
        ],
        mesh=mesh,
        scratch_types=[
            pltpu.VMEM((SROWS, 128), f32),
            pltpu.VMEM((SROWS, 128), f32),
            pltpu.VMEM((SROWS, 128), f32),
            pltpu.VMEM((DROWS, 128), f32),
            pltpu.VMEM((wrows, 128), f32),
            pltpu.VMEM((8, K), jnp.int32),
            pltpu.VMEM((8, K), jnp.int32),
            pltpu.VMEM((128,), f32),
            pltpu.VMEM((128,), jnp.int32),
            pltpu.VMEM((32,), jnp.int32),
            pltpu.VMEM((DROWS,), jnp.int32),
            pltpu.VMEM_SHARED((SROWS, 128), f32),
            pltpu.VMEM_SHARED((DROWS, 128), f32),
        ],
        compiler_params=scp,
    )(asrc_s, adst_s, gmax128, rowi1, coli1)

    s4 = jnp.concatenate(
        [s_out[0].reshape(N_T, 2), s_out[1].reshape(N_T, 2)], axis=1)
    deg = deg_out.reshape(N_T, 1)

    nsu = pl.kernel(
        functools.partial(_sc1b_body, c1),
        out_type=jax.ShapeDtypeStruct((NC, N_T, 128), f32),
        mesh=mesh,
        scratch_types=[
            pltpu.VMEM((8, K), jnp.int32),
            pltpu.VMEM((8, K), jnp.int32),
            pltpu.VMEM((K, 128), f32),
            pltpu.VMEM((K, 128), f32),
            pltpu.VMEM((8, 128), f32),
            pltpu.VMEM((8, 128), f32),
            pltpu.SemaphoreType.DMA,
            pltpu.SemaphoreType.DMA,
            pltpu.VMEM_SHARED((N_T, 128), f32),
        ],
        compiler_params=scp,
    )(xs_t, xl_t, w_e, rowi1, coli1)
    ns, u_acc = nsu[0], nsu[1]

    gw1p = jnp.zeros((1, 128), f32).at[0, :16].set(gate_w1[0])
    gb1p = jnp.zeros((1, 128), f32).at[0, :16].set(gate_b1)
    gw2p = jnp.zeros((1, 128), f32).at[0, :16].set(gate_w2[:, 0])
    ew = jnp.zeros((128, 128), f32).at[:, :42].set(ego_w)
    hw = jnp.zeros((128, 128), f32).at[:, 42:84].set(hop_w0)
    bb = jnp.zeros((1, 128), f32).at[0, :42].set(ego_b).at[0, 42:84].set(
        hop_b0)
    as4 = jnp.concatenate([as01, as23], axis=1)
    ad4 = jnp.concatenate([ad01, ad23], axis=1)

    h1t, dsb, gb, hh, sb, uself = pl.pallas_call(
        _tc2_body,
        grid=(grid,),
        in_specs=[
            _row_spec(blk, 128),
            _row_spec(blk, 1),
            _row_spec(blk, 4),
            _row_spec(blk, 4),
            _row_spec(blk, 4),
            _row_spec(blk, 128),
            _row_spec(blk, 128),
            _full_spec((1, 16)),
            _full_spec((1, 128)),
            _full_spec((1, 128)),
            _full_spec((1, 128)),
            _full_spec((1, 1)),
            _full_spec((128, 128)),
            _full_spec((128, 128)),
            _full_spec((1, 128)),
        ],
        out_specs=[_row_spec(blk, 128)] * 6,
        out_shape=[jax.ShapeDtypeStruct((N_T, 128), f32)] * 6,
    )(ns, deg, s4, as4, ad4, xs_t, xl_t, gmax128[:16].reshape(1, 16),
      gw1p, gb1p, gw2p, gate_b2.reshape(1, 1), ew, hw, bb)

    ns2 = pl.kernel(
        functools.partial(_sc1c_body, c2),
        out_type=jax.ShapeDtypeStruct((NC, N_T, 128), f32),
        mesh=mesh,
        scratch_types=[
            pltpu.VMEM((8, K), jnp.int32),
            pltpu.VMEM((8, K), jnp.int32),
            pltpu.VMEM((K, 128), f32),
            pltpu.VMEM((K, 128), f32),
            pltpu.SemaphoreType.DMA,
            pltpu.SemaphoreType.DMA,
            pltpu.VMEM_SHARED((N_T, 128), f32),
        ],
        compiler_params=scp,
    )(h1t, rowi2, coli2)

    hw1 = jnp.zeros((128, 128), f32).at[:, 84:128].set(hop_w1)
    b1p = jnp.zeros((1, 128), f32).at[0, 84:128].set(hop_b1)
    clswp = jnp.zeros((128, 128), f32).at[:, :2].set(cls_w)
    clsbp = jnp.zeros((1, 128), f32).at[0, :2].set(cls_b)

    out = pl.pallas_call(
        _tc3_body,
        grid=(grid,),
        in_specs=[
            _row_spec(blk, 128),
            _row_spec(blk, 128),
            _row_spec(blk, 128),
            _row_spec(blk, 128),
            _row_spec(blk, 128),
            _row_spec(blk, 128),
            _row_spec(blk, 128),
            _row_spec(blk, 128),
            _full_spec((1, 128)),
            _full_spec((128, 128)),
            _full_spec((1, 128)),
            _full_spec((1, 128)),
            _full_spec((1, 128)),
            _full_spec((1, 128)),
            _full_spec((1, 128)),
            _full_spec((128, 128)),
            _full_spec((1, 128)),
        ],
        out_specs=_row_spec(blk, 128),
        out_shape=jax.ShapeDtypeStruct((N_T, 128), f32),
    )(ns2[0], ns2[1], dsb, hh, gb, u_acc, uself, sb,
      gat_bias.reshape(1, 128), hw1, b1p,
      bn_gamma.reshape(1, 128), bn_beta.reshape(1, 128),
      bn_mean.reshape(1, 128), bn_var.reshape(1, 128), clswp, clsbp)

    return out[:n, :2]


# speedup vs baseline: 24.1841x; 1.0084x over previous
import functools

import jax
import jax.numpy as jnp
from jax import lax
from jax.experimental import pallas as pl
from jax.experimental.pallas import tpu as pltpu
from jax.experimental.pallas import tpu_sc as plsc

N_T = 10240
N1 = 10016
K = 128
NC = 2
NS = 16
RPT = N_T // NS
SROWS = 160
DROWS = 80


def _full16(v):
    return jnp.full((16,), v, jnp.int32)


def _tc1_body(n, x_ref, fi_ref, gw_ref, was_ref, wad_ref,
              xs_ref, xl_ref, as01_ref, as23_ref, ad01_ref, ad23_ref,
              pmax_ref):
    xs = x_ref[:] * jax.nn.sigmoid(fi_ref[:])
    xs_ref[:] = xs
    xl = jnp.dot(xs, gw_ref[:], preferred_element_type=jnp.float32)
    xl_ref[:] = xl
    asrc16 = jnp.dot(xl, was_ref[:], preferred_element_type=jnp.float32)
    ad16 = jnp.dot(xl, wad_ref[:], preferred_element_type=jnp.float32)
    as01_ref[:] = asrc16[:, 0:2]
    as23_ref[:] = asrc16[:, 2:4]
    ad01_ref[:] = ad16[:, 0:2]
    ad23_ref[:] = ad16[:, 2:4]
    pa = jnp.max(asrc16, axis=0, keepdims=True)
    pb = jnp.max(ad16, axis=0, keepdims=True)
    pmax_ref[0] = jnp.concatenate(
        [pa, pb, jnp.zeros((1, 96), jnp.float32)], axis=1)
    del n


def _sc1a_body(c1, asrc_s, adst_s, gmax_hbm, rowi_hbm, coli_hbm,
               w_out, s_out, deg_out,
               as_v, ad_v, s_acc, deg_acc, wall, rowi8, coli8, gv,
               idxb, idxb2, idxd, sp_s, sp_deg):
    cid = lax.axis_index("c")
    sid = lax.axis_index("s")
    i32 = jnp.int32
    z16 = jnp.zeros((16,), jnp.float32)
    iota16 = lax.iota(i32, 16)

    def _zs(r, c):
        for s in range(8):
            s_acc[r, pl.ds(s * 16, 16)] = z16
        return c
    lax.fori_loop(0, SROWS, _zs, 0)

    def _zd(r, c):
        for s in range(8):
            deg_acc[r, pl.ds(s * 16, 16)] = z16
        return c
    lax.fori_loop(0, DROWS, _zd, 0)

    @pl.when(sid < 10)
    def _():
        pltpu.sync_copy(s_acc.at[pl.ds(0, 16)],
                        sp_s.at[pl.ds(sid * 16, 16)])

    @pl.when(jnp.logical_and(cid == 0, sid < 10))
    def _():
        pltpu.sync_copy(deg_acc.at[pl.ds(0, 8)],
                        sp_deg.at[pl.ds(sid * 8, 8)])

    pltpu.sync_copy(asrc_s.at[cid], as_v)
    pltpu.sync_copy(adst_s.at[cid], ad_v)
    pltpu.sync_copy(gmax_hbm, gv)

    g0 = plsc.load_gather(gv, [_full16(2 * cid)])
    g1 = plsc.load_gather(gv, [_full16(2 * cid + 1)])
    mask2 = iota16 < 2
    maskl0 = iota16 < 1
    onesv = jnp.ones((16,), jnp.float32)

    def blk8(j8, carry):
        pltpu.sync_copy(rowi_hbm.at[sid, pl.ds(j8 * 8, 8)], rowi8)
        pltpu.sync_copy(coli_hbm.at[sid, pl.ds(j8 * 8, 8)], coli8)

        def chunk(jj, cc):
            j = j8 * 8 + jj

            def qloop(q, qc):
                rows16 = rowi8[jj, pl.ds(q * 16, 16)]
                cols16 = coli8[jj, pl.ds(q * 16, 16)]
                fr = rows16 * 2
                fc = cols16 * 2
                a0 = plsc.load_gather(as_v, [fr >> 7, fr & 127]) + \
                    plsc.load_gather(ad_v, [fc >> 7, fc & 127])
                a1 = plsc.load_gather(as_v, [(fr + 1) >> 7, (fr + 1) & 127]) \
                    + plsc.load_gather(ad_v, [(fc + 1) >> 7, (fc + 1) & 127])
                w0 = jnp.exp(jnp.maximum(a0, 0.2 * a0) - g0)
                w1 = jnp.exp(jnp.maximum(a1, 0.2 * a1) - g1)
                fw = (j * K + q * 16 + iota16) * 2
                plsc.store_scatter(wall, [fw >> 7, fw & 127], w0)
                fw1 = fw + 1
                plsc.store_scatter(wall, [fw1 >> 7, fw1 & 127], w1)
                return qc
            lax.fori_loop(0, K // 16, qloop, 0)

            @pl.when(cid == 0)
            def _():
                def eloop(t, ec):
                    for u in range(4):
                        e = t * 4 + u
                        ef = _full16(e)
                        jf = _full16(jj)
                        cv = plsc.load_gather(coli8, [jf, ef])
                        flat = cv * 2 + iota16
                        fw = _full16((j * K + e) * 2) + iota16
                        wv = plsc.load_gather(
                            wall, [fw >> 7, fw & 127], mask=mask2)
                        plsc.addupdate_scatter(
                            s_acc, [flat >> 7, flat & 127], wv, mask=mask2)
                        rv = plsc.load_gather(rowi8, [jf, ef])
                        plsc.addupdate_scatter(
                            deg_acc, [rv >> 7, rv & 127], onesv,
                            mask=maskl0)
                    return ec
                lax.fori_loop(0, K // 4, eloop, 0)

            @pl.when(cid != 0)
            def _():
                def eloop(t, ec):
                    for u in range(4):
                        e = t * 4 + u
                        ef = _full16(e)
                        jf = _full16(jj)
                        cv = plsc.load_gather(coli8, [jf, ef])
                        flat = cv * 2 + iota16
                        fw = _full16((j * K + e) * 2) + iota16
                        wv = plsc.load_gather(
                            wall, [fw >> 7, fw & 127], mask=mask2)
                        plsc.addupdate_scatter(
                            s_acc, [flat >> 7, flat & 127], wv, mask=mask2)
                    return ec
                lax.fori_loop(0, K // 4, eloop, 0)
            return cc
        lax.fori_loop(0, 8, chunk, 0)
        return carry
    lax.fori_loop(0, c1 // 8, blk8, 0)

    pltpu.sync_copy(wall, w_out.at[cid, sid])
    plsc.subcore_barrier()

    def fill(buf, nrows, base):
        for kk in range(nrows // 16):
            buf[pl.ds(kk * 16, 16)] = iota16 + (base + kk * 16)
    fill(idxb, 128, 0)
    fill(idxb2, 32, 128)
    pltpu.sync_copy(s_acc.at[pl.ds(0, 128)], sp_s.at[idxb], add=True)
    pltpu.sync_copy(s_acc.at[pl.ds(128, 32)], sp_s.at[idxb2], add=True)

    @pl.when(cid == 0)
    def _():
        fill(idxd, DROWS, 0)
        pltpu.sync_copy(deg_acc, sp_deg.at[idxd], add=True)

    plsc.subcore_barrier()

    @pl.when(sid < 10)
    def _():
        pltpu.sync_copy(sp_s.at[pl.ds(sid * 16, 16)],
                        s_out.at[cid, pl.ds(sid * 16, 16)])

    @pl.when(jnp.logical_and(cid == 0, sid < 10))
    def _():
        pltpu.sync_copy(sp_deg.at[pl.ds(sid * 8, 8)],
                        deg_out.at[pl.ds(sid * 8, 8)])


def _sc1b_body(c1, xs_hbm, xl_hbm, w_hbm, rowi_hbm, coli_hbm,
               nsu_out,
               rowi8, coli8, dbuf0, dbuf1, wch0, wch1, sem0, sem1,
               ssem0, ssem1, acc):
    cid = lax.axis_index("c")
    sid = lax.axis_index("s")
    z16 = jnp.zeros((16,), jnp.float32)
    bufs = (dbuf0, dbuf1)
    sems = (sem0, sem1)
    ssems = (ssem0, ssem1)

    def _zrow(r, c):
        for s in range(8):
            dbuf0[r, pl.ds(s * 16, 16)] = z16
        return c
    lax.fori_loop(0, K, _zrow, 0)
    for q in range(RPT // K):
        pltpu.sync_copy(dbuf0, acc.at[pl.ds(sid * RPT + q * K, K)])
    plsc.subcore_barrier()

    @pl.when(cid == 0)
    def _():
        def blk8(j8, carry):
            pltpu.sync_copy(rowi_hbm.at[sid, pl.ds(j8 * 8, 8)], rowi8)
            pltpu.sync_copy(coli_hbm.at[sid, pl.ds(j8 * 8, 8)], coli8)
            d = pltpu.async_copy(xs_hbm.at[coli8.at[0]], bufs[0], sems[0])
            dS = [None, None]
            for jj in range(8):
                p = jj & 1
                if jj < 7:
                    if dS[1 - p] is not None:
                        dS[1 - p].wait()
                    dn = pltpu.async_copy(
                        xs_hbm.at[coli8.at[jj + 1]], bufs[1 - p],
                        sems[1 - p])
                d.wait()
                dS[p] = pltpu.async_copy(
                    bufs[p], acc.at[rowi8.at[jj]], ssems[p], add=True)
                if jj < 7:
                    d = dn
            dS[0].wait()
            dS[1].wait()
            return carry
        lax.fori_loop(0, c1 // 8, blk8, 0)

    @pl.when(cid != 0)
    def _():
        def blk8(j8, carry):
            pltpu.sync_copy(rowi_hbm.at[sid, pl.ds(j8 * 8, 8)], rowi8)
            pltpu.sync_copy(coli_hbm.at[sid, pl.ds(j8 * 8, 8)], coli8)
            d = pltpu.async_copy(xl_hbm.at[rowi8.at[0]], bufs[0], sems[0])
            dS = [None, None]
            for jj in range(8):
                p = jj & 1
                if jj % 4 == 0:
                    g4 = jj // 4
                    pltpu.sync_copy(
                        w_hbm.at[0, sid, pl.ds((j8 * 2 + g4) * 8, 8)],
                        wch0)
                    pltpu.sync_copy(
                        w_hbm.at[1, sid, pl.ds((j8 * 2 + g4) * 8, 8)],
                        wch1)
                if jj < 7:
                    if dS[1 - p] is not None:
                        dS[1 - p].wait()
                    dn = pltpu.async_copy(
                        xl_hbm.at[rowi8.at[jj + 1]], bufs[1 - p],
                        sems[1 - p])
                d.wait()
                db = bufs[p]
                ljk = (jj % 4) * K

                def edge(t, ec):
                    for u in range(2):
                        e = t * 2 + u
                        lf = _full16((ljk + e) * 2)
                        s0 = plsc.load_gather(wch0, [lf >> 7, lf & 127])
                        lf1 = lf + 1
                        s1 = plsc.load_gather(wch0, [lf1 >> 7, lf1 & 127])
                        s2 = plsc.load_gather(wch1, [lf >> 7, lf & 127])
                        s3 = plsc.load_gather(wch1, [lf1 >> 7, lf1 & 127])
                        for v, sc in ((0, s0), (1, s0), (2, s1), (3, s1),
                                      (4, s2), (5, s2), (6, s3), (7, s3)):
                            db[e, pl.ds(v * 16, 16)] = \
                                db[e, pl.ds(v * 16, 16)] * sc
                    return ec
                lax.fori_loop(0, K // 2, edge, 0)
                dS[p] = pltpu.async_copy(
                    db, acc.at[coli8.at[jj]], ssems[p], add=True)
                if jj < 7:
                    d = dn
            dS[0].wait()
            dS[1].wait()
            return carry
        lax.fori_loop(0, c1 // 8, blk8, 0)

    plsc.subcore_barrier()
    sl = pl.ds(sid * RPT, RPT)
    pltpu.sync_copy(acc.at[sl], nsu_out.at[cid].at[sl])


def _sc1c_body(c2, h1_hbm, rowi_hbm, coli_hbm, ns2_out,
               rowi8, coli8, dbuf0, dbuf1, sem0, sem1, ssem0, ssem1, acc):
    cid = lax.axis_index("c")
    sid = lax.axis_index("s")
    wid = sid * NC + cid
    z16 = jnp.zeros((16,), jnp.float32)
    bufs = (dbuf0, dbuf1)
    sems = (sem0, sem1)
    ssems = (ssem0, ssem1)

    def _zrow(r, c):
        for s in range(8):
            dbuf0[r, pl.ds(s * 16, 16)] = z16
        return c
    lax.fori_loop(0, K, _zrow, 0)
    for q in range(RPT // K):
        pltpu.sync_copy(dbuf0, acc.at[pl.ds(sid * RPT + q * K, K)])
    plsc.subcore_barrier()

    def blk8(j8, carry):
        pltpu.sync_copy(rowi_hbm.at[wid, pl.ds(j8 * 8, 8)], rowi8)
        pltpu.sync_copy(coli_hbm.at[wid, pl.ds(j8 * 8, 8)], coli8)
        d = pltpu.async_copy(h1_hbm.at[coli8.at[0]], bufs[0], sems[0])
        dS = [None, None]
        for jj in range(8):
            p = jj & 1
            if jj < 7:
                if dS[1 - p] is not None:
                    dS[1 - p].wait()
                dn = pltpu.async_copy(
                    h1_hbm.at[coli8.at[jj + 1]], bufs[1 - p], sems[1 - p])
            d.wait()
            dS[p] = pltpu.async_copy(
                bufs[p], acc.at[rowi8.at[jj]], ssems[p], add=True)
            if jj < 7:
                d = dn
        dS[0].wait()
        dS[1].wait()
        return carry
    lax.fori_loop(0, c2 // 8, blk8, 0)

    plsc.subcore_barrier()
    sl = pl.ds(sid * RPT, RPT)
    pltpu.sync_copy(acc.at[sl], ns2_out.at[cid].at[sl])


def _tc2_body(ns_ref, deg_ref, s4_ref, as4_ref, ad4_ref, xs_ref, xl_ref,
              gmax_ref, gw1_ref, gb1_ref, gw2_ref, gb2_ref,
              ew_ref, hw_ref, bb_ref,
              h1_ref, ds_ref, g_ref, hh_ref, sb_ref, uself_ref):
    blk = ns_ref.shape[0]
    nsum = ns_ref[:]
    deg = deg_ref[:]
    deg_safe = jnp.maximum(deg, 1.0)
    h1 = nsum / deg_safe
    h1_ref[:] = h1
    ds_ref[:] = jnp.broadcast_to(1.0 / deg_safe, (blk, 128))

    xs = xs_ref[:]
    xn = xs / jnp.maximum(
        jnp.sqrt(jnp.sum(xs * xs, axis=1, keepdims=True)), 1e-12)
    mn = h1 / jnp.maximum(
        jnp.sqrt(jnp.sum(h1 * h1, axis=1, keepdims=True)), 1e-12)
    sim = jnp.sum(xn * mn, axis=1, keepdims=True)
    sim = jnp.where(deg > 0, sim, 1.0)
    delta = jax.nn.sigmoid(deg * (1.0 - sim) / 10.0 - 0.5)

    g16 = jax.nn.relu(delta * gw1_ref[:] + gb1_ref[:])
    gs = jnp.sum(g16 * gw2_ref[:], axis=1, keepdims=True) + gb2_ref[0, 0]
    g = jax.nn.sigmoid(gs)
    g_ref[:] = jnp.broadcast_to(g, (blk, 128))

    s = as4_ref[:] + ad4_ref[:]
    alpha = jnp.maximum(s, 0.2 * s)
    w_self = jnp.exp(alpha - gmax_ref[:, :4])
    s4 = s4_ref[:] + w_self
    rmat = (lax.broadcasted_iota(jnp.int32, (4, 128), 1) // 32
            == lax.broadcasted_iota(jnp.int32, (4, 128), 0)
            ).astype(jnp.float32)
    sb_ref[:] = jnp.dot(s4, rmat, preferred_element_type=jnp.float32)
    wb = jnp.dot(w_self, rmat, preferred_element_type=jnp.float32)
    uself_ref[:] = wb * xl_ref[:]

    hh = jnp.dot(xs, ew_ref[:], preferred_element_type=jnp.float32)
    hh = hh + jnp.dot(h1, hw_ref[:], preferred_element_type=jnp.float32)
    hh_ref[:] = jax.nn.relu(hh + bb_ref[:])


def _tc3_body(p0_ref, p1_ref, ds_ref, hh_ref, g_ref, u_ref, uself_ref,
              sb_ref, gbias_ref, hw1_ref, b1_ref,
              bng_ref, bnb_ref, bnm_ref, bnv_ref, clsw_ref, clsb_ref,
              out_ref):
    hlp = (u_ref[:] + uself_ref[:]) / (sb_ref[:] + 1e-16) + gbias_ref[:]
    h_low = jnp.where(hlp > 0, hlp, jnp.exp(jnp.minimum(hlp, 0.0)) - 1.0)
    h2 = (p0_ref[:] + p1_ref[:]) * ds_ref[:]
    f1 = jax.nn.relu(
        jnp.dot(h2, hw1_ref[:], preferred_element_type=jnp.float32)
        + b1_ref[:])
    h_high = hh_ref[:] + f1
    g = g_ref[:]
    h = (1.0 - g) * h_low + g * h_high
    h = (h - bnm_ref[:]) / jnp.sqrt(bnv_ref[:] + 1e-5) * bng_ref[:] \
        + bnb_ref[:]
    out_ref[:] = jnp.dot(h, clsw_ref[:], preferred_element_type=jnp.float32) \
        + clsb_ref[:]


def _row_spec(blk, w):
    return pl.BlockSpec((blk, w), lambda i: (i, 0))


def _full_spec(shape):
    return pl.BlockSpec(shape, lambda i: tuple(0 for _ in shape))


def kernel(x, edge_index, feat_imp, gat_w, gat_att_src, gat_att_dst,
           gat_bias, ego_w, ego_b, hop_w0, hop_b0, hop_w1, hop_b1,
           gate_w1, gate_b1, gate_w2, gate_b2,
           bn_gamma, bn_beta, bn_mean, bn_var, cls_w, cls_b):
    n = x.shape[0]
    e = edge_index.shape[1]
    f32 = jnp.float32

    x_pad = jnp.zeros((N_T, 128), f32).at[:n].set(x)
    lane128 = jnp.arange(128)
    was = jnp.zeros((128, 16), f32).at[lane128, lane128 // 32].set(
        gat_att_src.reshape(128))
    wad = jnp.zeros((128, 16), f32).at[lane128, lane128 // 32].set(
        gat_att_dst.reshape(128))

    row = edge_index[0]
    col = edge_index[1]
    c1 = -(-e // (NS * K))
    c1 = -(-c1 // 8) * 8
    e1 = c1 * NS * K
    rowi1 = jnp.concatenate([row, jnp.full((e1 - e,), n, jnp.int32)]
                            ).reshape(NS, c1, K)
    coli1 = jnp.concatenate([col, jnp.full((e1 - e,), n, jnp.int32)]
                            ).reshape(NS, c1, K)
    c2 = -(-e // (NC * NS * K))
    c2 = -(-c2 // 8) * 8
    e2 = c2 * NC * NS * K
    rowi2 = jnp.concatenate([row, jnp.full((e2 - e,), n, jnp.int32)]
                            ).reshape(NC * NS, c2, K)
    coli2 = jnp.concatenate([col, jnp.full((e2 - e,), n, jnp.int32)]
                            ).reshape(NC * NS, c2, K)

    blk = 256
    grid = N_T // blk

    xs_t, xl_t, as01, as23, ad01, ad23, pmax = pl.pallas_call(
        functools.partial(_tc1_body, n),
        grid=(grid,),
        in_specs=[
            _row_spec(blk, 128),
            _full_spec((1, 128)),
            _full_spec((128, 128)),
            _full_spec((128, 16)),
            _full_spec((128, 16)),
        ],
        out_specs=[
            _row_spec(blk, 128),
            _row_spec(blk, 128),
            _row_spec(blk, 2),
            _row_spec(blk, 2),
            _row_spec(blk, 2),
            _row_spec(blk, 2),
            pl.BlockSpec((1, 1, 128), lambda i: (i, 0, 0)),
        ],
        out_shape=[
            jax.ShapeDtypeStruct((N_T, 128), f32),
            jax.ShapeDtypeStruct((N_T, 128), f32),
            jax.ShapeDtypeStruct((N_T, 2), f32),
            jax.ShapeDtypeStruct((N_T, 2), f32),
            jax.ShapeDtypeStruct((N_T, 2), f32),
            jax.ShapeDtypeStruct((N_T, 2), f32),
            jax.ShapeDtypeStruct((grid, 1, 128), f32),
        ],
    )(x_pad, feat_imp.reshape(1, 128), gat_w, was, wad)

    pm = jnp.max(pmax[:, 0, :], axis=0)
    gsum = pm[0:4] + pm[16:20]
    gmax4 = jnp.where(gsum > 0, gsum, 0.2 * gsum)
    gmax128 = jnp.zeros((128,), f32).at[:4].set(gmax4)
    asrc_s = jnp.stack([as01.reshape(SROWS, 128), as23.reshape(SROWS, 128)])
    adst_s = jnp.stack([ad01.reshape(SROWS, 128), ad23.reshape(SROWS, 128)])

    mesh = plsc.VectorSubcoreMesh(core_axis_name="c", subcore_axis_name="s")
    scp = pltpu.CompilerParams(needs_layout_passes=False)
    wrows = c1 * 2

    w_e, s_out, deg_out = pl.kernel(
        functools.partial(_sc1a_body, c1),
        out_type=[
            jax.ShapeDtypeStruct((NC, NS, wrows, 128), f32),
            jax.ShapeDtypeStruct((NC, SROWS, 128), f32),
            jax.ShapeDtypeStruct((DROWS, 128), f32),
        ],
        mesh=mesh,
        scratch_types=[
            pltpu.VMEM((SROWS, 128), f32),
            pltpu.VMEM((SROWS, 128), f32),
            pltpu.VMEM((SROWS, 128), f32),
            pltpu.VMEM((DROWS, 128), f32),
            pltpu.VMEM((wrows, 128), f32),
            pltpu.VMEM((8, K), jnp.int32),
            pltpu.VMEM((8, K), jnp.int32),
            pltpu.VMEM((128,), f32),
            pltpu.VMEM((128,), jnp.int32),
            pltpu.VMEM((32,), jnp.int32),
            pltpu.VMEM((DROWS,), jnp.int32),
            pltpu.VMEM_SHARED((SROWS, 128), f32),
            pltpu.VMEM_SHARED((DROWS, 128), f32),
        ],
        compiler_params=scp,
    )(asrc_s, adst_s, gmax128, rowi1, coli1)

    s4 = jnp.concatenate(
        [s_out[0].reshape(N_T, 2), s_out[1].reshape(N_T, 2)], axis=1)
    deg = deg_out.reshape(N_T, 1)

    nsu = pl.kernel(
        functools.partial(_sc1b_body, c1),
        out_type=jax.ShapeDtypeStruct((NC, N_T, 128), f32),
        mesh=mesh,
        scratch_types=[
            pltpu.VMEM((8, K), jnp.int32),
            pltpu.VMEM((8, K), jnp.int32),
            pltpu.VMEM((K, 128), f32),
            pltpu.VMEM((K, 128), f32),
            pltpu.VMEM((8, 128), f32),
            pltpu.VMEM((8, 128), f32),
            pltpu.SemaphoreType.DMA,
            pltpu.SemaphoreType.DMA,
            pltpu.SemaphoreType.DMA,
            pltpu.SemaphoreType.DMA,
            pltpu.VMEM_SHARED((N_T, 128), f32),
        ],
        compiler_params=scp,
    )(xs_t, xl_t, w_e, rowi1, coli1)
    ns, u_acc = nsu[0], nsu[1]

    gw1p = jnp.zeros((1, 128), f32).at[0, :16].set(gate_w1[0])
    gb1p = jnp.zeros((1, 128), f32).at[0, :16].set(gate_b1)
    gw2p = jnp.zeros((1, 128), f32).at[0, :16].set(gate_w2[:, 0])
    ew = jnp.zeros((128, 128), f32).at[:, :42].set(ego_w)
    hw = jnp.zeros((128, 128), f32).at[:, 42:84].set(hop_w0)
    bb = jnp.zeros((1, 128), f32).at[0, :42].set(ego_b).at[0, 42:84].set(
        hop_b0)
    as4 = jnp.concatenate([as01, as23], axis=1)
    ad4 = jnp.concatenate([ad01, ad23], axis=1)

    h1t, dsb, gb, hh, sb, uself = pl.pallas_call(
        _tc2_body,
        grid=(grid,),
        in_specs=[
            _row_spec(blk, 128),
            _row_spec(blk, 1),
            _row_spec(blk, 4),
            _row_spec(blk, 4),
            _row_spec(blk, 4),
            _row_spec(blk, 128),
            _row_spec(blk, 128),
            _full_spec((1, 16)),
            _full_spec((1, 128)),
            _full_spec((1, 128)),
            _full_spec((1, 128)),
            _full_spec((1, 1)),
            _full_spec((128, 128)),
            _full_spec((128, 128)),
            _full_spec((1, 128)),
        ],
        out_specs=[_row_spec(blk, 128)] * 6,
        out_shape=[jax.ShapeDtypeStruct((N_T, 128), f32)] * 6,
    )(ns, deg, s4, as4, ad4, xs_t, xl_t, gmax128[:16].reshape(1, 16),
      gw1p, gb1p, gw2p, gate_b2.reshape(1, 1), ew, hw, bb)

    ns2 = pl.kernel(
        functools.partial(_sc1c_body, c2),
        out_type=jax.ShapeDtypeStruct((NC, N_T, 128), f32),
        mesh=mesh,
        scratch_types=[
            pltpu.VMEM((8, K), jnp.int32),
            pltpu.VMEM((8, K), jnp.int32),
            pltpu.VMEM((K, 128), f32),
            pltpu.VMEM((K, 128), f32),
            pltpu.SemaphoreType.DMA,
            pltpu.SemaphoreType.DMA,
            pltpu.SemaphoreType.DMA,
            pltpu.SemaphoreType.DMA,
            pltpu.VMEM_SHARED((N_T, 128), f32),
        ],
        compiler_params=scp,
    )(h1t, rowi2, coli2)

    hw1 = jnp.zeros((128, 128), f32).at[:, 84:128].set(hop_w1)
    b1p = jnp.zeros((1, 128), f32).at[0, 84:128].set(hop_b1)
    clswp = jnp.zeros((128, 128), f32).at[:, :2].set(cls_w)
    clsbp = jnp.zeros((1, 128), f32).at[0, :2].set(cls_b)

    out = pl.pallas_call(
        _tc3_body,
        grid=(grid,),
        in_specs=[
            _row_spec(blk, 128),
            _row_spec(blk, 128),
            _row_spec(blk, 128),
            _row_spec(blk, 128),
            _row_spec(blk, 128),
            _row_spec(blk, 128),
            _row_spec(blk, 128),
            _row_spec(blk, 128),
            _full_spec((1, 128)),
            _full_spec((128, 128)),
            _full_spec((1, 128)),
            _full_spec((1, 128)),
            _full_spec((1, 128)),
            _full_spec((1, 128)),
            _full_spec((1, 128)),
            _full_spec((128, 128)),
            _full_spec((1, 128)),
        ],
        out_specs=_row_spec(blk, 128),
        out_shape=jax.ShapeDtypeStruct((N_T, 128), f32),
    )(ns2[0], ns2[1], dsb, hh, gb, u_acc, uself, sb,
      gat_bias.reshape(1, 128), hw1, b1p,
      bn_gamma.reshape(1, 128), bn_beta.reshape(1, 128),
      bn_mean.reshape(1, 128), bn_var.reshape(1, 128), clswp, clsbp)

    return out[:n, :2]
